# baseline (speedup 1.0000x reference)
import functools

import jax
import jax.numpy as jnp
from jax.experimental import pallas as pl
from jax.experimental.pallas import tpu as pltpu

_MARGIN = 1.0
_BIG = 4096.0


def _triplet_block(x_ref, y_ref, out_ref, lhs_ref, rhs_ref, *, br, n):
    i = pl.program_id(0)

    @pl.when(i == 0)
    def _prep():
        xr = x_ref[...].astype(jnp.bfloat16)
        xf = xr.astype(jnp.float32)
        x2 = jnp.sum(xf * xf, axis=1, keepdims=True)
        x2h = x2.astype(jnp.bfloat16).astype(jnp.float32)
        x2l = x2 - x2h
        classes = jax.lax.broadcasted_iota(jnp.int32, (n, 112), 1)
        eqf = jnp.where(y_ref[...] == classes, 1.0, 0.0)
        is_h = (classes == 100).astype(jnp.float32)
        is_l = (classes == 101).astype(jnp.float32)
        ohl = (eqf + is_h + is_l).astype(jnp.bfloat16)
        ohr = (eqf * _BIG + is_h * x2h + is_l * x2l).astype(jnp.bfloat16)
        lhs_ref[...] = jnp.concatenate([jnp.bfloat16(-2.0) * xr, ohl], axis=1)
        rhs_ref[...] = jnp.concatenate([xr, ohr], axis=1)

    u = jax.lax.dot_general(
        lhs_ref[pl.ds(i * br, br), :], rhs_ref[...],
        dimension_numbers=(((1,), (1,)), ((), ())),
        preferred_element_type=jnp.float32,
    )
    mx = jnp.max(u, axis=1)
    mn = jnp.min(u, axis=1)
    per = jax.nn.relu(mx - (_BIG - _MARGIN) - mn)
    partial = jnp.sum(per).reshape(1, 1)

    @pl.when(i == 0)
    def _init():
        out_ref[...] = jnp.zeros((1, 1), jnp.float32)

    out_ref[...] += partial


def kernel(x, y):
    n, d = x.shape
    br = 1024
    grid = n // br

    out = pl.pallas_call(
        functools.partial(_triplet_block, br=br, n=n),
        grid=(grid,),
        in_specs=[
            pl.BlockSpec((n, d), lambda i: (0, 0)),
            pl.BlockSpec((n, 1), lambda i: (0, 0)),
        ],
        out_specs=pl.BlockSpec((1, 1), lambda i: (0, 0)),
        out_shape=jax.ShapeDtypeStruct((1, 1), jnp.float32),
        scratch_shapes=[
            pltpu.VMEM((n, d + 112), jnp.bfloat16),
            pltpu.VMEM((n, d + 112), jnp.bfloat16),
        ],
    )(x, y.reshape(n, 1))
    return out[0, 0] / n


# speedup vs baseline: 1.1046x; 1.1046x over previous
import functools

import jax
import jax.numpy as jnp
from jax.experimental import pallas as pl
from jax.experimental.pallas import tpu as pltpu

_MARGIN = 1.0
_BIG = 4096.0


def _triplet_block(x_ref, y_ref, out_ref, lhs_ref, rhs_ref, *, br, n):
    i = pl.program_id(0)

    @pl.when(i == 0)
    def _prep():
        xr = x_ref[...].astype(jnp.bfloat16)
        xf = xr.astype(jnp.float32)
        x2 = jnp.sum(xf * xf, axis=1, keepdims=True)
        x2h = x2.astype(jnp.bfloat16).astype(jnp.float32)
        x2l = x2 - x2h
        classes = jax.lax.broadcasted_iota(jnp.int32, (n, 112), 1)
        eqf = jnp.where(y_ref[...] == classes, 1.0, 0.0)
        is_h = (classes == 100).astype(jnp.float32)
        is_l = (classes == 101).astype(jnp.float32)
        ohl = (eqf + is_h + is_l).astype(jnp.bfloat16)
        ohr = (eqf * _BIG + is_h * x2h + is_l * x2l).astype(jnp.bfloat16)
        lhs_ref[...] = jnp.concatenate([jnp.bfloat16(-2.0) * xr, ohl], axis=1)
        rhs_ref[...] = jnp.concatenate([xr, ohr], axis=1)

    u = jax.lax.dot_general(
        lhs_ref[pl.ds(i * br, br), :], rhs_ref[...],
        dimension_numbers=(((1,), (1,)), ((), ())),
        preferred_element_type=jnp.float32,
    )
    mx = jnp.max(u, axis=1)
    mn = jnp.min(u, axis=1)
    per = jax.nn.relu(mx - (_BIG - _MARGIN) - mn)
    partial = (jnp.sum(per) / n).reshape(1, 1)

    @pl.when(i == 0)
    def _init():
        out_ref[...] = jnp.zeros((1, 1), jnp.float32)

    out_ref[...] += partial


def kernel(x, y):
    n, d = x.shape
    br = 2048
    grid = n // br

    out = pl.pallas_call(
        functools.partial(_triplet_block, br=br, n=n),
        grid=(grid,),
        in_specs=[
            pl.BlockSpec((n, d), lambda i: (0, 0)),
            pl.BlockSpec((n, 1), lambda i: (0, 0)),
        ],
        out_specs=pl.BlockSpec((1, 1), lambda i: (0, 0)),
        out_shape=jax.ShapeDtypeStruct((1, 1), jnp.float32),
        scratch_shapes=[
            pltpu.VMEM((n, d + 112), jnp.bfloat16),
            pltpu.VMEM((n, d + 112), jnp.bfloat16),
        ],
    )(x, y.reshape(n, 1))
    return out[0, 0]
